# baseline (speedup 1.0000x reference)
import jax
import jax.numpy as jnp
from jax.experimental import pallas as pl

GAIN_TANH = 1.5927812
RSQRT2 = 0.7071067811865476


def _fermi_block(hp_ref, t0_ref, t1_ref, wbig_ref, w2_ref, w3_ref, bbig_ref,
                 wp0_ref, bp0_ref, wp1_ref, bp1_ref,
                 ho_ref, o0_ref, o1_ref):
    hp = hp_ref[...]
    t0 = t0_ref[...]
    t1 = t1_ref[...]
    u = jnp.dot(hp, wbig_ref[...], preferred_element_type=jnp.float32)
    u += jnp.dot(t0, w2_ref[...], preferred_element_type=jnp.float32)
    u += jnp.dot(t1, w3_ref[...], preferred_element_type=jnp.float32)
    u += bbig_ref[...]
    ho_ref[...] = (hp + jnp.tanh(u * RSQRT2) * GAIN_TANH) * RSQRT2

    v0 = jnp.dot(t0, wp0_ref[...], preferred_element_type=jnp.float32) + bp0_ref[...]
    o0_ref[...] = (t0 + jnp.tanh(v0) * GAIN_TANH) * RSQRT2
    v1 = jnp.dot(t1, wp1_ref[...], preferred_element_type=jnp.float32) + bp1_ref[...]
    o1_ref[...] = (t1 + jnp.tanh(v1) * GAIN_TANH) * RSQRT2


def kernel(h_one, h_two_0, h_two_1, spins, W_single, b_single, W_global,
           W_pair0, b_pair0, W_pair1, b_pair1):
    N, d_one = h_one.shape
    d_pair = h_two_0.shape[1]
    G = N // 2
    D = 2 * d_one
    P = 2 * d_pair

    Ws1 = W_single[:d_one]
    Ws2 = W_single[d_one:d_one + d_pair]
    Ws3 = W_single[d_one + d_pair:]
    Wg_top = W_global[:d_one]
    Wg_bot = W_global[d_one:]
    Wa = Ws1 + Wg_top
    zeros_pair = jnp.zeros((d_pair, d_one), jnp.float32)
    W_big = jnp.block([[Wa, Wg_bot], [Wg_bot, Wa]])
    W2_big = jnp.block([[Ws2, zeros_pair], [zeros_pair, Ws2]])
    W3_big = jnp.block([[Ws3, zeros_pair], [zeros_pair, Ws3]])
    b_big = jnp.tile(b_single, 2).reshape(1, D)
    zp = jnp.zeros((d_pair, d_pair), jnp.float32)
    Wp0_big = jnp.block([[W_pair0, zp], [zp, W_pair0]])
    Wp1_big = jnp.block([[W_pair1, zp], [zp, W_pair1]])
    bp0_big = jnp.tile(b_pair0, 2).reshape(1, P)
    bp1_big = jnp.tile(b_pair1, 2).reshape(1, P)

    hp = h_one.reshape(G, D)
    t0p = h_two_0.reshape(G, P)
    t1p = h_two_1.reshape(G, P)

    R = 4096
    grid = (G // R,)

    row_spec = lambda w: pl.BlockSpec((R, w), lambda i: (i, 0))
    full_spec = lambda a: pl.BlockSpec(a.shape, lambda i: (0, 0))

    ho, o0, o1 = pl.pallas_call(
        _fermi_block,
        grid=grid,
        in_specs=[
            row_spec(D), row_spec(P), row_spec(P),
            full_spec(W_big), full_spec(W2_big), full_spec(W3_big),
            full_spec(b_big),
            full_spec(Wp0_big), full_spec(bp0_big),
            full_spec(Wp1_big), full_spec(bp1_big),
        ],
        out_specs=[row_spec(D), row_spec(P), row_spec(P)],
        out_shape=[
            jax.ShapeDtypeStruct((G, D), jnp.float32),
            jax.ShapeDtypeStruct((G, P), jnp.float32),
            jax.ShapeDtypeStruct((G, P), jnp.float32),
        ],
    )(hp, t0p, t1p, W_big, W2_big, W3_big, b_big,
      Wp0_big, bp0_big, Wp1_big, bp1_big)

    return (ho.reshape(N, d_one), o0.reshape(N, d_pair), o1.reshape(N, d_pair))


# speedup vs baseline: 1.7565x; 1.7565x over previous
import jax
import jax.numpy as jnp
from jax.experimental import pallas as pl

GAIN_TANH = 1.5927812
RSQRT2 = 0.7071067811865476


def _fermi_block(x_ref, t0_ref, t1_ref, wa_ref, wb_ref, w2_ref, w3_ref, b_ref,
                 wp0_ref, bp0_ref, wp1_ref, bp1_ref,
                 ho_ref, o0_ref, o1_ref):
    x = x_ref[...]
    t0 = t0_ref[...]
    t1 = t1_ref[...]

    parity = jax.lax.broadcasted_iota(jnp.int32, (x.shape[0], 1), 0) % 2
    xs = jnp.where(parity == 0, jnp.roll(x, -1, axis=0), jnp.roll(x, 1, axis=0))

    u = jnp.dot(x, wa_ref[...], preferred_element_type=jnp.float32)
    u += jnp.dot(xs, wb_ref[...], preferred_element_type=jnp.float32)
    u += jnp.dot(t0, w2_ref[...], preferred_element_type=jnp.float32)
    u += jnp.dot(t1, w3_ref[...], preferred_element_type=jnp.float32)
    u += b_ref[...]
    ho_ref[...] = (x + jnp.tanh(u * RSQRT2) * GAIN_TANH) * RSQRT2

    v0 = jnp.dot(t0, wp0_ref[...], preferred_element_type=jnp.float32) + bp0_ref[...]
    o0_ref[...] = (t0 + jnp.tanh(v0) * GAIN_TANH) * RSQRT2
    v1 = jnp.dot(t1, wp1_ref[...], preferred_element_type=jnp.float32) + bp1_ref[...]
    o1_ref[...] = (t1 + jnp.tanh(v1) * GAIN_TANH) * RSQRT2


def kernel(h_one, h_two_0, h_two_1, spins, W_single, b_single, W_global,
           W_pair0, b_pair0, W_pair1, b_pair1):
    N, d_one = h_one.shape
    d_pair = h_two_0.shape[1]

    Wa = W_single[:d_one] + W_global[:d_one]
    Wb = W_global[d_one:]
    Ws2 = W_single[d_one:d_one + d_pair]
    Ws3 = W_single[d_one + d_pair:]
    b = b_single.reshape(1, d_one)
    bp0 = b_pair0.reshape(1, d_pair)
    bp1 = b_pair1.reshape(1, d_pair)

    T = 4096
    grid = (N // T,)

    row_spec = lambda w: pl.BlockSpec((T, w), lambda i: (i, 0))
    full_spec = lambda a: pl.BlockSpec(a.shape, lambda i: (0, 0))

    ho, o0, o1 = pl.pallas_call(
        _fermi_block,
        grid=grid,
        in_specs=[
            row_spec(d_one), row_spec(d_pair), row_spec(d_pair),
            full_spec(Wa), full_spec(Wb), full_spec(Ws2), full_spec(Ws3),
            full_spec(b),
            full_spec(W_pair0), full_spec(bp0),
            full_spec(W_pair1), full_spec(bp1),
        ],
        out_specs=[row_spec(d_one), row_spec(d_pair), row_spec(d_pair)],
        out_shape=[
            jax.ShapeDtypeStruct((N, d_one), jnp.float32),
            jax.ShapeDtypeStruct((N, d_pair), jnp.float32),
            jax.ShapeDtypeStruct((N, d_pair), jnp.float32),
        ],
    )(h_one, h_two_0, h_two_1, Wa, Wb, Ws2, Ws3, b,
      W_pair0, bp0, W_pair1, bp1)

    return (ho, o0, o1)
